# baseline (speedup 1.0000x reference)
import functools

import jax
import jax.numpy as jnp
from jax import lax
from jax.experimental import pallas as pl
from jax.experimental.pallas import tpu as pltpu
from jax.experimental.pallas import tpu_sc as plsc

_N = 512
_K = 1024
_D = 256
_NCAND = 4
_B = _NCAND * _N
_NC = 2
_NS = 16
_NW = _NC * _NS
_BPW = _B // _NW


def _topk_kernel(xt_ref, cb_ref, st_ref, cand_ref):
    xt = xt_ref[...]
    cb = cb_ref[...]
    xct = lax.dot_general(
        cb, xt,
        dimension_numbers=(((1,), (0,)), ((), ())),
        preferred_element_type=jnp.float32,
        precision=lax.Precision.HIGHEST,
    )
    cb2 = cb * cb
    ones = jnp.ones((_D, 1), jnp.float32)
    cnorm = lax.dot_general(
        cb2, ones,
        dimension_numbers=(((1,), (0,)), ((), ())),
        preferred_element_type=jnp.float32,
        precision=lax.Precision.HIGHEST,
    )
    st = cnorm - 2.0 * xct
    riota = lax.broadcasted_iota(jnp.int32, (_K, _N), 0)
    rows = []
    for _ in range(_NCAND):
        m = jnp.min(st, axis=0, keepdims=True)
        i = jnp.min(jnp.where(st == m, riota, _K), axis=0,
                    keepdims=True).astype(jnp.int32)
        rows.append(i)
        st = jnp.where(riota == i, jnp.inf, st)
    st_ref[...] = jnp.float32(0)[None, None]
    cand_ref[...] = jnp.concatenate(rows, axis=0)


def _sc_gather_body(cb_hbm, idx_hbm, out_hbm, idx_v, rows_v, sem):
    wid = lax.axis_index("s") * _NC + lax.axis_index("c")
    base = wid * _BPW
    pltpu.sync_copy(idx_hbm.at[pl.ds(base, _BPW)], idx_v)
    pltpu.async_copy(cb_hbm.at[idx_v], rows_v, sem).wait()
    pltpu.sync_copy(rows_v, out_hbm.at[pl.ds(base, _BPW)])


@functools.cache
def _sc_gather():
    mesh = plsc.VectorSubcoreMesh(
        core_axis_name="c", subcore_axis_name="s", num_cores=_NC,
        num_subcores=_NS)
    return pl.kernel(
        _sc_gather_body,
        out_type=jax.ShapeDtypeStruct((_B, _D), jnp.float32),
        mesh=mesh,
        scratch_types=[
            pltpu.VMEM((_BPW,), jnp.int32),
            pltpu.VMEM((_BPW, _D), jnp.float32),
            pltpu.SemaphoreType.DMA,
        ],
    )


def _refined_dist(xt, rowt):
    sq = (xt - rowt) * (xt - rowt)
    totals = []
    for c in range(2):
        r = sq[128 * c:128 * (c + 1), :].reshape(16, 8, _N)
        p = r[0]
        for k in range(1, 16):
            p = p + r[k]
        a = ((p[0:1] + p[4:5]) + (p[2:3] + p[6:7])) + (
            (p[1:2] + p[5:6]) + (p[3:4] + p[7:8]))
        totals.append(a)
    return (totals[0] + totals[1]) * jnp.float32(1.0 / _D)


def _refine_kernel(xt_ref, rows_ref, cand_ref, loss_ref, qt_ref, idx_ref):
    xt = xt_ref[...]
    best_d = None
    best_i = None
    best_qt = None
    for c in range(_NCAND):
        rowt = jnp.transpose(rows_ref[c], (1, 0))
        i_c = cand_ref[c:c + 1, :]
        d = _refined_dist(xt, rowt)
        if best_d is None:
            best_d, best_i, best_qt = d, i_c, rowt
        else:
            lt = (d < best_d) | ((d == best_d) & (i_c < best_i))
            best_d = jnp.where(lt, d, best_d)
            best_i = jnp.where(lt, i_c, best_i)
            best_qt = jnp.where(lt, rowt, best_qt)
    diff = best_qt - xt
    m2 = jnp.sum(diff * diff) * jnp.float32(1.0 / (_N * _D))
    loss_ref[...] = (m2 + jnp.float32(0.25) * m2)[None, None]
    qt_ref[...] = xt + (best_qt - xt)
    idx_ref[...] = best_i


@jax.jit
def kernel(inputs, codebook):
    xt = jnp.transpose(inputs, (0, 2, 3, 1)).reshape(_N, _D).T
    _, cand = pl.pallas_call(
        _topk_kernel,
        out_shape=(
            jax.ShapeDtypeStruct((1, 1), jnp.float32),
            jax.ShapeDtypeStruct((_NCAND, _N), jnp.int32),
        ),
    )(xt, codebook)
    rows = _sc_gather()(codebook, cand.reshape(_B))
    loss, qt, idx = pl.pallas_call(
        _refine_kernel,
        out_shape=(
            jax.ShapeDtypeStruct((1, 1), jnp.float32),
            jax.ShapeDtypeStruct((_D, _N), jnp.float32),
            jax.ShapeDtypeStruct((1, _N), jnp.int32),
        ),
    )(xt, rows.reshape(_NCAND, _N, _D), cand)
    quantized = jnp.transpose(qt.reshape(_D, 2, 16, 16), (1, 0, 2, 3))
    return loss.reshape(()), quantized, idx.reshape(2, 256)


# speedup vs baseline: 2.2514x; 2.2514x over previous
import jax
import jax.numpy as jnp
from jax import lax
from jax.experimental import pallas as pl

_N = 512
_K = 1024
_D = 256
_NCAND = 4


def _bdot(a, b, dims):
    return lax.dot_general(a, b, dimension_numbers=(dims, ((), ())),
                           preferred_element_type=jnp.float32)


def _refined_dist(xt, rowt):
    sq = (xt - rowt) * (xt - rowt)
    totals = []
    for c in range(2):
        r = sq[128 * c:128 * (c + 1), :].reshape(16, 8, _N)
        p = r[0]
        for k in range(1, 16):
            p = p + r[k]
        a = ((p[0:1] + p[4:5]) + (p[2:3] + p[6:7])) + (
            (p[1:2] + p[5:6]) + (p[3:4] + p[7:8]))
        totals.append(a)
    return (totals[0] + totals[1]) * jnp.float32(1.0 / _D)


def _vq_kernel(in_ref, cb_ref, loss_ref, q_ref, idx_ref):
    xt = jnp.concatenate([in_ref[0], in_ref[1]], axis=1)
    cb = cb_ref[...]

    cb1 = cb.astype(jnp.bfloat16)
    r1 = cb - cb1.astype(jnp.float32)
    cb2 = r1.astype(jnp.bfloat16)
    r2 = r1 - cb2.astype(jnp.float32)
    cb3 = r2.astype(jnp.bfloat16)

    xt1 = xt.astype(jnp.bfloat16)
    xt2 = (xt - xt1.astype(jnp.float32)).astype(jnp.bfloat16)
    cdims = (((1,), (0,)), ((), ()))
    xct = (lax.dot_general(cb1, xt1, dimension_numbers=cdims,
                           preferred_element_type=jnp.float32)
           + lax.dot_general(cb1, xt2, dimension_numbers=cdims,
                             preferred_element_type=jnp.float32)
           + lax.dot_general(cb2, xt1, dimension_numbers=cdims,
                             preferred_element_type=jnp.float32))
    cb2sq = cb * cb
    sq1 = cb2sq.astype(jnp.bfloat16)
    sq2 = (cb2sq - sq1.astype(jnp.float32)).astype(jnp.bfloat16)
    ones = jnp.ones((_D, 1), jnp.bfloat16)
    cnorm = (lax.dot_general(sq1, ones, dimension_numbers=cdims,
                             preferred_element_type=jnp.float32)
             + lax.dot_general(sq2, ones, dimension_numbers=cdims,
                               preferred_element_type=jnp.float32))
    st = cnorm - 2.0 * xct

    riota = lax.broadcasted_iota(jnp.int32, (_K, _N), 0)
    qs = lax.convert_element_type(st * jnp.float32(1 << 20), jnp.int32)
    packed = ((qs + jnp.int32(1 << 20)) << 10) | riota
    cand_idx = []
    for _ in range(_NCAND):
        mp = jnp.min(packed, axis=0, keepdims=True)
        i = jnp.bitwise_and(mp, jnp.int32(_K - 1))
        cand_idx.append(i)
        packed = jnp.where(packed == mp, jnp.int32(0x7fffffff), packed)

    gdims = (((0,), (0,)), ((), ()))
    oht4 = jnp.concatenate(
        [(riota == i).astype(jnp.bfloat16) for i in cand_idx], axis=1)
    rowt4 = (_bdot(cb1, oht4, gdims[0]) + _bdot(cb2, oht4, gdims[0])
             ) + _bdot(cb3, oht4, gdims[0])
    best_d = None
    best_i = None
    best_qt = None
    for c in range(_NCAND):
        i_c = cand_idx[c]
        rowt = rowt4[:, c * _N:(c + 1) * _N]
        d = _refined_dist(xt, rowt)
        if best_d is None:
            best_d, best_i, best_qt = d, i_c, rowt
        else:
            lt = (d < best_d) | ((d == best_d) & (i_c < best_i))
            best_d = jnp.where(lt, d, best_d)
            best_i = jnp.where(lt, i_c, best_i)
            best_qt = jnp.where(lt, rowt, best_qt)

    diff = best_qt - xt
    m2 = jnp.sum(diff * diff) * jnp.float32(1.0 / (_N * _D))
    loss_ref[...] = (m2 + jnp.float32(0.25) * m2)[None, None]
    qst = xt + (best_qt - xt)
    q_ref[...] = jnp.stack([qst[:, :_N // 2], qst[:, _N // 2:]], axis=0)
    idx_ref[...] = best_i


@jax.jit
def kernel(inputs, codebook):
    flat = inputs.reshape(2, _D, 256)
    loss, q, idx = pl.pallas_call(
        _vq_kernel,
        out_shape=(
            jax.ShapeDtypeStruct((1, 1), jnp.float32),
            jax.ShapeDtypeStruct((2, _D, 256), jnp.float32),
            jax.ShapeDtypeStruct((1, _N), jnp.int32),
        ),
    )(flat, codebook)
    quantized = q.reshape(2, _D, 16, 16)
    return loss.reshape(()), quantized, idx.reshape(2, 256)
